# Initial kernel scaffold; baseline (speedup 1.0000x reference)
import jax
import jax.numpy as jnp
from jax.experimental import pallas as pl


def kernel(x, pos_table):
    raise NotImplementedError("write your pallas kernel here")


# speedup vs baseline: 1.4509x; 1.4509x over previous
import jax
import jax.numpy as jnp
from jax.experimental import pallas as pl

_BLOCK_S = 256


def _body(x_ref, pos_ref, out_ref):
    out_ref[...] = x_ref[...] + pos_ref[...][None]


def kernel(x, pos_table):
    B, S, D = x.shape
    grid = (S // _BLOCK_S, B)
    return pl.pallas_call(
        _body,
        grid=grid,
        in_specs=[
            pl.BlockSpec((1, _BLOCK_S, D), lambda i, b: (b, i, 0)),
            pl.BlockSpec((_BLOCK_S, D), lambda i, b: (i, 0)),
        ],
        out_specs=pl.BlockSpec((1, _BLOCK_S, D), lambda i, b: (b, i, 0)),
        out_shape=jax.ShapeDtypeStruct((B, S, D), x.dtype),
    )(x, pos_table)


# speedup vs baseline: 1.9227x; 1.3251x over previous
import jax
import jax.numpy as jnp
from jax.experimental import pallas as pl

_BLOCK_S = 512


def _body(x_ref, pos_ref, out_ref):
    out_ref[...] = x_ref[...] + pos_ref[...][None]


def kernel(x, pos_table):
    B, S, D = x.shape
    grid = (S // _BLOCK_S, B)
    return pl.pallas_call(
        _body,
        grid=grid,
        in_specs=[
            pl.BlockSpec((1, _BLOCK_S, D), lambda i, b: (b, i, 0)),
            pl.BlockSpec((_BLOCK_S, D), lambda i, b: (i, 0)),
        ],
        out_specs=pl.BlockSpec((1, _BLOCK_S, D), lambda i, b: (b, i, 0)),
        out_shape=jax.ShapeDtypeStruct((B, S, D), x.dtype),
    )(x, pos_table)


# speedup vs baseline: 2.1022x; 1.0934x over previous
import jax
import jax.numpy as jnp
from jax.experimental import pallas as pl

_BLOCK_S = 1024


def _body(x_ref, pos_ref, out_ref):
    out_ref[...] = x_ref[...] + pos_ref[...][None]


def kernel(x, pos_table):
    B, S, D = x.shape
    grid = (S // _BLOCK_S, B)
    return pl.pallas_call(
        _body,
        grid=grid,
        in_specs=[
            pl.BlockSpec((1, _BLOCK_S, D), lambda i, b: (b, i, 0)),
            pl.BlockSpec((_BLOCK_S, D), lambda i, b: (i, 0)),
        ],
        out_specs=pl.BlockSpec((1, _BLOCK_S, D), lambda i, b: (b, i, 0)),
        out_shape=jax.ShapeDtypeStruct((B, S, D), x.dtype),
    )(x, pos_table)


# speedup vs baseline: 2.2898x; 1.0892x over previous
import jax
import jax.numpy as jnp
from jax.experimental import pallas as pl

_BLOCK_S = 2048


def _body(x_ref, pos_ref, out_ref):
    out_ref[...] = x_ref[...] + pos_ref[...][None]


def kernel(x, pos_table):
    B, S, D = x.shape
    grid = (S // _BLOCK_S, B)
    return pl.pallas_call(
        _body,
        grid=grid,
        in_specs=[
            pl.BlockSpec((1, _BLOCK_S, D), lambda i, b: (b, i, 0)),
            pl.BlockSpec((_BLOCK_S, D), lambda i, b: (i, 0)),
        ],
        out_specs=pl.BlockSpec((1, _BLOCK_S, D), lambda i, b: (b, i, 0)),
        out_shape=jax.ShapeDtypeStruct((B, S, D), x.dtype),
    )(x, pos_table)
